# baseline (speedup 1.0000x reference)
import functools

import jax
import jax.numpy as jnp
from jax import lax
from jax.experimental import pallas as pl
from jax.experimental.pallas import tpu as pltpu
from jax.experimental.pallas import tpu_sc as plsc

_N = 10000
_E = 320000
_D = 128
_NC = 2
_NS = 16
_HALF = _D // _NC
_CHUNK = 128
_NCHUNK = 157
_EPT = _NCHUNK * _CHUNK
_EPAD = _NS * _EPT - _E
_NACC = _N + 16
_RPT = 624
_RREM = _N - _NS * _RPT
_NB = 4
_AHEAD = 2
_BODY = 152
_ZROWS = _NB * _CHUNK


def _sc_body(x2, e4, out, sidx, didx, rows, acc, *sems):
    gsem = sems[:_NB]
    ssem = sems[_NB:]
    c = lax.axis_index("c")
    s = lax.axis_index("s")
    r0 = s * _RPT
    xv = x2.at[c]

    zvec = jnp.zeros((16,), jnp.float32)

    @pl.loop(0, _ZROWS)
    def _zero_rows(i):
        for j in range(_HALF // 16):
            rows[i, pl.ds(j * 16, 16)] = zvec

    for h in range(2):
        pltpu.sync_copy(
            rows.at[pl.ds(0, _RPT // 2)],
            acc.at[pl.ds(r0 + h * (_RPT // 2), _RPT // 2)],
        )

    @pl.when(s == 0)
    def _zero_rem():
        pltpu.sync_copy(
            rows.at[pl.ds(0, _NACC - _NS * _RPT)],
            acc.at[pl.ds(_NS * _RPT, _NACC - _NS * _RPT)],
        )

    plsc.subcore_barrier()

    pltpu.sync_copy(e4.at[0].at[s], sidx)
    pltpu.sync_copy(e4.at[1].at[s], didx)

    def rbuf(b):
        return rows.at[pl.ds(b * _CHUNK, _CHUNK)]

    def gather(k, b):
        pltpu.async_copy(xv.at[sidx.at[k]], rbuf(b), gsem[b])

    def gather_wait(k, b):
        pltpu.make_async_copy(xv.at[sidx.at[k]], rbuf(b), gsem[b]).wait()

    def scatter(k, b):
        pltpu.async_copy(rbuf(b), acc.at[didx.at[k]], ssem[b], add=True)

    def scatter_drain(b):
        pltpu.make_async_copy(x2.at[0].at[pl.ds(0, _CHUNK)], rbuf(b), ssem[b]).wait()

    for k in range(_AHEAD):
        gather(k, k % _NB)

    @pl.loop(0, _BODY, step=_NB)
    def _chunks(i):
        for b in range(_NB):
            k = i + b
            bg = (b + _AHEAD) % _NB
            if b < _AHEAD:
                @pl.when(k >= _AHEAD)
                def _drain():
                    scatter_drain(bg)

            else:
                scatter_drain(bg)
            gather(k + _AHEAD, bg)
            gather_wait(k, b)
            scatter(k, b)

    for k in range(_BODY, _NCHUNK - _AHEAD):
        b = k % _NB
        bg = (b + _AHEAD) % _NB
        scatter_drain(bg)
        gather(k + _AHEAD, bg)
        gather_wait(k, b)
        scatter(k, b)

    for k in range(_NCHUNK - _AHEAD, _NCHUNK):
        b = k % _NB
        gather_wait(k, b)
        scatter(k, b)

    for k in range(_NCHUNK - _NB, _NCHUNK):
        scatter_drain(k % _NB)

    plsc.subcore_barrier()

    pltpu.sync_copy(
        acc.at[pl.ds(r0, _RPT)],
        out.at[pl.ds(r0, _RPT), pl.ds(c * _HALF, _HALF)],
    )

    @pl.when(s == 0)
    def _write_rem():
        pltpu.sync_copy(
            acc.at[pl.ds(_NS * _RPT, _RREM)],
            out.at[pl.ds(_NS * _RPT, _RREM), pl.ds(c * _HALF, _HALF)],
        )


_mp_kernel = functools.partial(
    pl.kernel,
    out_type=jax.ShapeDtypeStruct((_N, _D), jnp.float32),
    mesh=plsc.VectorSubcoreMesh(core_axis_name="c", subcore_axis_name="s"),
    scratch_types=[
        pltpu.VMEM((_NCHUNK, _CHUNK), jnp.int32),
        pltpu.VMEM((_NCHUNK, _CHUNK), jnp.int32),
        pltpu.VMEM((_ZROWS, _HALF), jnp.float32),
        pltpu.VMEM_SHARED((_NACC, _HALF), jnp.float32),
    ]
    + [pltpu.SemaphoreType.DMA] * (2 * _NB),
    compiler_params=pltpu.CompilerParams(use_tc_tiling_on_sc=False),
)(_sc_body)


def kernel(x, edge_index):
    x2 = x.reshape(_N, _NC, _HALF).transpose(1, 0, 2)
    pad = jnp.stack(
        [jnp.zeros((_EPAD,), jnp.int32), jnp.full((_EPAD,), _N, jnp.int32)]
    )
    e4 = jnp.concatenate([edge_index, pad], axis=1)
    e4 = e4.reshape(_NC, _NS, _NCHUNK, _CHUNK)
    return _mp_kernel(x2, e4)


# speedup vs baseline: 1.2831x; 1.2831x over previous
import functools

import jax
import jax.numpy as jnp
from jax import lax
from jax.experimental import pallas as pl
from jax.experimental.pallas import tpu as pltpu
from jax.experimental.pallas import tpu_sc as plsc

_N = 10000
_E = 320000
_D = 128
_NC = 2
_NS = 16
_HALF = _D // _NC
_CHUNK = 80
_NCHUNK = 250
_NACC = _N
_RPT = 624
_RREM = _N - _NS * _RPT
_NB = 4
_AHEAD = 2
_BODY = 248
_ZROWS = _NB * _CHUNK


def _sc_body(x2, e4, out, sidx, didx, rows, acc, *sems):
    gsem = sems[:_NB]
    ssem = sems[_NB:]
    c = lax.axis_index("c")
    s = lax.axis_index("s")
    r0 = s * _RPT
    xv = x2.at[c]

    zvec = jnp.zeros((16,), jnp.float32)

    @pl.loop(0, _ZROWS)
    def _zero_rows(i):
        for j in range(_HALF // 16):
            rows[i, pl.ds(j * 16, 16)] = zvec

    for h in range(2):
        pltpu.sync_copy(
            rows.at[pl.ds(0, _RPT // 2)],
            acc.at[pl.ds(r0 + h * (_RPT // 2), _RPT // 2)],
        )

    @pl.when(s == 0)
    def _zero_rem():
        pltpu.sync_copy(
            rows.at[pl.ds(0, _NACC - _NS * _RPT)],
            acc.at[pl.ds(_NS * _RPT, _NACC - _NS * _RPT)],
        )

    plsc.subcore_barrier()

    pltpu.sync_copy(e4.at[0].at[s], sidx)
    pltpu.sync_copy(e4.at[1].at[s], didx)

    def rbuf(b):
        return rows.at[pl.ds(b * _CHUNK, _CHUNK)]

    def gather(k, b):
        pltpu.async_copy(xv.at[sidx.at[k]], rbuf(b), gsem[b])

    def gather_wait(k, b):
        pltpu.make_async_copy(xv.at[sidx.at[k]], rbuf(b), gsem[b]).wait()

    def scatter(k, b):
        pltpu.async_copy(rbuf(b), acc.at[didx.at[k]], ssem[b], add=True)

    def scatter_drain(b):
        pltpu.make_async_copy(x2.at[0].at[pl.ds(0, _CHUNK)], rbuf(b), ssem[b]).wait()

    for k in range(_AHEAD):
        gather(k, k % _NB)

    @pl.loop(0, _BODY, step=_NB)
    def _chunks(i):
        for b in range(_NB):
            k = i + b
            bg = (b + _AHEAD) % _NB
            if b < _AHEAD:
                @pl.when(k >= _AHEAD)
                def _drain():
                    scatter_drain(bg)

            else:
                scatter_drain(bg)
            gather(k + _AHEAD, bg)
            gather_wait(k, b)
            scatter(k, b)

    for k in range(_BODY, _NCHUNK - _AHEAD):
        b = k % _NB
        bg = (b + _AHEAD) % _NB
        scatter_drain(bg)
        gather(k + _AHEAD, bg)
        gather_wait(k, b)
        scatter(k, b)

    for k in range(_NCHUNK - _AHEAD, _NCHUNK):
        b = k % _NB
        gather_wait(k, b)
        scatter(k, b)

    for k in range(_NCHUNK - _NB, _NCHUNK):
        scatter_drain(k % _NB)

    plsc.subcore_barrier()

    pltpu.sync_copy(
        acc.at[pl.ds(r0, _RPT)],
        out.at[pl.ds(r0, _RPT), pl.ds(c * _HALF, _HALF)],
    )

    @pl.when(s == 0)
    def _write_rem():
        pltpu.sync_copy(
            acc.at[pl.ds(_NS * _RPT, _RREM)],
            out.at[pl.ds(_NS * _RPT, _RREM), pl.ds(c * _HALF, _HALF)],
        )


_mp_kernel = functools.partial(
    pl.kernel,
    out_type=jax.ShapeDtypeStruct((_N, _D), jnp.float32),
    mesh=plsc.VectorSubcoreMesh(core_axis_name="c", subcore_axis_name="s"),
    scratch_types=[
        pltpu.VMEM((_NCHUNK, _CHUNK), jnp.int32),
        pltpu.VMEM((_NCHUNK, _CHUNK), jnp.int32),
        pltpu.VMEM((_ZROWS, _HALF), jnp.float32),
        pltpu.VMEM_SHARED((_NACC, _HALF), jnp.float32),
    ]
    + [pltpu.SemaphoreType.DMA] * (2 * _NB),
    compiler_params=pltpu.CompilerParams(use_tc_tiling_on_sc=False),
)(_sc_body)


def kernel(x, edge_index):
    x2 = x.reshape(_N, _NC, _HALF).transpose(1, 0, 2)
    e4 = edge_index.reshape(_NC, _NS, _NCHUNK, _CHUNK)
    return _mp_kernel(x2, e4)


# speedup vs baseline: 1.4070x; 1.0965x over previous
import functools

import jax
import jax.numpy as jnp
from jax import lax
from jax.experimental import pallas as pl
from jax.experimental.pallas import tpu as pltpu
from jax.experimental.pallas import tpu_sc as plsc

_N = 10000
_E = 320000
_D = 128
_NC = 2
_NS = 16
_HALF = _D // _NC
_CHUNK = 80
_NCHUNK = 250
_NACC = _N
_RPT = 624
_RREM = _N - _NS * _RPT
_NB = 6
_AHEAD = 3
_BODY = 246
_ZROWS = _NB * _CHUNK


def _sc_body(x2, e4, out, sidx, didx, rows, acc, *sems):
    gsem = sems[:_NB]
    ssem = sems[_NB:]
    c = lax.axis_index("c")
    s = lax.axis_index("s")
    r0 = s * _RPT
    xv = x2.at[c]

    zvec = jnp.zeros((16,), jnp.float32)

    @pl.loop(0, _ZROWS)
    def _zero_rows(i):
        for j in range(_HALF // 16):
            rows[i, pl.ds(j * 16, 16)] = zvec

    for h in range(2):
        pltpu.sync_copy(
            rows.at[pl.ds(0, _RPT // 2)],
            acc.at[pl.ds(r0 + h * (_RPT // 2), _RPT // 2)],
        )

    @pl.when(s == 0)
    def _zero_rem():
        pltpu.sync_copy(
            rows.at[pl.ds(0, _NACC - _NS * _RPT)],
            acc.at[pl.ds(_NS * _RPT, _NACC - _NS * _RPT)],
        )

    plsc.subcore_barrier()

    pltpu.sync_copy(e4.at[0].at[s], sidx)
    pltpu.sync_copy(e4.at[1].at[s], didx)

    def rbuf(b):
        return rows.at[pl.ds(b * _CHUNK, _CHUNK)]

    def gather(k, b):
        pltpu.async_copy(xv.at[sidx.at[k]], rbuf(b), gsem[b])

    def gather_wait(k, b):
        pltpu.make_async_copy(xv.at[sidx.at[k]], rbuf(b), gsem[b]).wait()

    def scatter(k, b):
        pltpu.async_copy(rbuf(b), acc.at[didx.at[k]], ssem[b], add=True)

    def scatter_drain(b):
        pltpu.make_async_copy(x2.at[0].at[pl.ds(0, _CHUNK)], rbuf(b), ssem[b]).wait()

    for k in range(_AHEAD):
        gather(k, k % _NB)

    @pl.loop(0, _BODY, step=_NB)
    def _chunks(i):
        for b in range(_NB):
            k = i + b
            bg = (b + _AHEAD) % _NB
            if b < _AHEAD:
                @pl.when(k >= _AHEAD)
                def _drain():
                    scatter_drain(bg)

            else:
                scatter_drain(bg)
            gather(k + _AHEAD, bg)
            gather_wait(k, b)
            scatter(k, b)

    for k in range(_BODY, _NCHUNK - _AHEAD):
        b = k % _NB
        bg = (b + _AHEAD) % _NB
        scatter_drain(bg)
        gather(k + _AHEAD, bg)
        gather_wait(k, b)
        scatter(k, b)

    for k in range(_NCHUNK - _AHEAD, _NCHUNK):
        b = k % _NB
        gather_wait(k, b)
        scatter(k, b)

    for k in range(_NCHUNK - _NB, _NCHUNK):
        scatter_drain(k % _NB)

    plsc.subcore_barrier()

    pltpu.sync_copy(
        acc.at[pl.ds(r0, _RPT)],
        out.at[pl.ds(r0, _RPT), pl.ds(c * _HALF, _HALF)],
    )

    @pl.when(s == 0)
    def _write_rem():
        pltpu.sync_copy(
            acc.at[pl.ds(_NS * _RPT, _RREM)],
            out.at[pl.ds(_NS * _RPT, _RREM), pl.ds(c * _HALF, _HALF)],
        )


_mp_kernel = functools.partial(
    pl.kernel,
    out_type=jax.ShapeDtypeStruct((_N, _D), jnp.float32),
    mesh=plsc.VectorSubcoreMesh(core_axis_name="c", subcore_axis_name="s"),
    scratch_types=[
        pltpu.VMEM((_NCHUNK, _CHUNK), jnp.int32),
        pltpu.VMEM((_NCHUNK, _CHUNK), jnp.int32),
        pltpu.VMEM((_ZROWS, _HALF), jnp.float32),
        pltpu.VMEM_SHARED((_NACC, _HALF), jnp.float32),
    ]
    + [pltpu.SemaphoreType.DMA] * (2 * _NB),
    compiler_params=pltpu.CompilerParams(use_tc_tiling_on_sc=False),
)(_sc_body)


def kernel(x, edge_index):
    x2 = x.reshape(_N, _NC, _HALF).transpose(1, 0, 2)
    e4 = edge_index.reshape(_NC, _NS, _NCHUNK, _CHUNK)
    return _mp_kernel(x2, e4)


# speedup vs baseline: 1.6213x; 1.1523x over previous
import functools

import jax
import jax.numpy as jnp
from jax import lax
from jax.experimental import pallas as pl
from jax.experimental.pallas import tpu as pltpu
from jax.experimental.pallas import tpu_sc as plsc

_N = 10000
_E = 320000
_D = 128
_NC = 2
_NS = 16
_HALF = _D // _NC
_CHUNK = 80
_NCHUNK = 250
_NACC = _N
_RPT = 624
_RREM = _N - _NS * _RPT
_NB = 6
_AHEAD = 3
_BODY = 246
_ZROWS = _NB * _CHUNK


def _sc_body(x2, e4, out, sidx, didx, rows, acc, *sems):
    gsem = sems[:_NB]
    ssem = sems[_NB:]
    c = lax.axis_index("c")
    s = lax.axis_index("s")
    r0 = s * _RPT

    zvec = jnp.zeros((16,), jnp.float32)

    @pl.loop(0, _ZROWS)
    def _zero_rows(i):
        for j in range(_HALF // 16):
            rows[i, pl.ds(j * 16, 16)] = zvec

    for h in range(2):
        pltpu.sync_copy(
            rows.at[pl.ds(0, _RPT // 2)],
            acc.at[pl.ds(r0 + h * (_RPT // 2), _RPT // 2)],
        )

    @pl.when(s == 0)
    def _zero_rem():
        pltpu.sync_copy(
            rows.at[pl.ds(0, _NACC - _NS * _RPT)],
            acc.at[pl.ds(_NS * _RPT, _NACC - _NS * _RPT)],
        )

    plsc.subcore_barrier()

    pltpu.sync_copy(e4.at[0].at[s], sidx)
    pltpu.sync_copy(e4.at[1].at[s], didx)

    def rbuf(b):
        return rows.at[pl.ds(b * _CHUNK, _CHUNK)]

    def gather(k, b):
        for j in range(_CHUNK // 16):
            sl = pl.ds(j * 16, 16)
            sidx[k, sl] = sidx[k, sl] * 2 + c
        pltpu.async_copy(x2.at[sidx.at[k]], rbuf(b), gsem[b])

    def gather_wait(k, b):
        pltpu.make_async_copy(x2.at[sidx.at[k]], rbuf(b), gsem[b]).wait()

    def scatter(k, b):
        pltpu.async_copy(rbuf(b), acc.at[didx.at[k]], ssem[b], add=True)

    def scatter_drain(b):
        pltpu.make_async_copy(x2.at[pl.ds(0, _CHUNK)], rbuf(b), ssem[b]).wait()

    for k in range(_AHEAD):
        gather(k, k % _NB)

    @pl.loop(0, _BODY, step=_NB)
    def _chunks(i):
        for b in range(_NB):
            k = i + b
            bg = (b + _AHEAD) % _NB
            if b < _AHEAD:
                @pl.when(k >= _AHEAD)
                def _drain():
                    scatter_drain(bg)

            else:
                scatter_drain(bg)
            gather(k + _AHEAD, bg)
            gather_wait(k, b)
            scatter(k, b)

    for k in range(_BODY, _NCHUNK - _AHEAD):
        b = k % _NB
        bg = (b + _AHEAD) % _NB
        scatter_drain(bg)
        gather(k + _AHEAD, bg)
        gather_wait(k, b)
        scatter(k, b)

    for k in range(_NCHUNK - _AHEAD, _NCHUNK):
        b = k % _NB
        gather_wait(k, b)
        scatter(k, b)

    for k in range(_NCHUNK - _NB, _NCHUNK):
        scatter_drain(k % _NB)

    plsc.subcore_barrier()

    pltpu.sync_copy(
        acc.at[pl.ds(r0, _RPT)],
        out.at[pl.ds(r0, _RPT), pl.ds(c * _HALF, _HALF)],
    )

    @pl.when(s == 0)
    def _write_rem():
        pltpu.sync_copy(
            acc.at[pl.ds(_NS * _RPT, _RREM)],
            out.at[pl.ds(_NS * _RPT, _RREM), pl.ds(c * _HALF, _HALF)],
        )


_mp_kernel = functools.partial(
    pl.kernel,
    out_type=jax.ShapeDtypeStruct((_N, _D), jnp.float32),
    mesh=plsc.VectorSubcoreMesh(core_axis_name="c", subcore_axis_name="s"),
    scratch_types=[
        pltpu.VMEM((_NCHUNK, _CHUNK), jnp.int32),
        pltpu.VMEM((_NCHUNK, _CHUNK), jnp.int32),
        pltpu.VMEM((_ZROWS, _HALF), jnp.float32),
        pltpu.VMEM_SHARED((_NACC, _HALF), jnp.float32),
    ]
    + [pltpu.SemaphoreType.DMA] * (2 * _NB),
    compiler_params=pltpu.CompilerParams(use_tc_tiling_on_sc=False),
)(_sc_body)


def kernel(x, edge_index):
    x2 = x.reshape(_N * _NC, _HALF)
    e4 = edge_index.reshape(_NC, _NS, _NCHUNK, _CHUNK)
    return _mp_kernel(x2, e4)
